# baseline (speedup 1.0000x reference)
import functools

import jax
import jax.numpy as jnp
from jax import lax
from jax.experimental import pallas as pl
from jax.experimental.pallas import tpu as pltpu
from jax.experimental.pallas import tpu_sc as plsc

_BUF_ROWS = 56
_NBUF = 2


@functools.lru_cache(maxsize=None)
def _build(seq_len: int, channels: int, dtype_name: str):
    info = plsc.get_sparse_core_info()
    nw = info.num_cores * info.num_subcores
    assert seq_len % nw == 0
    rows_per_w = seq_len // nw
    sizes = []
    left = rows_per_w
    while left > 0:
        c = min(_BUF_ROWS, left)
        sizes.append(c)
        left -= c
    sizes.sort()
    starts = [sum(sizes[:i]) for i in range(len(sizes))]
    nchunks = len(sizes)
    dtype = jnp.dtype(dtype_name)
    mesh = plsc.VectorSubcoreMesh(core_axis_name="c", subcore_axis_name="s")

    def body(w_hbm, out_hbm, buf, load_sem, store_sem):
        wid = lax.axis_index("s") * info.num_cores + lax.axis_index("c")
        base = wid * rows_per_w

        def load(i, b):
            return pltpu.async_copy(
                w_hbm.at[pl.ds(base + starts[i], sizes[i])],
                buf.at[pl.ds(b * _BUF_ROWS, sizes[i])],
                load_sem.at[b])

        def store(i, b):
            return pltpu.async_copy(
                buf.at[pl.ds(b * _BUF_ROWS, sizes[i])],
                out_hbm.at[pl.ds(base + starts[i], sizes[i])],
                store_sem.at[b])

        loads = [None] * _NBUF
        stores = [None] * _NBUF
        loads[0] = load(0, 0)
        for i in range(nchunks):
            b = i % _NBUF
            nb = (i + 1) % _NBUF
            if i + 1 < nchunks:
                if stores[nb] is not None:
                    stores[nb].wait()
                loads[nb] = load(i + 1, nb)
            loads[b].wait()
            stores[b] = store(i, b)
        for b in range(_NBUF):
            if stores[b] is not None:
                stores[b].wait()

    return pl.kernel(
        body,
        out_type=jax.ShapeDtypeStruct((seq_len, channels), dtype),
        mesh=mesh,
        scratch_types=[
            pltpu.VMEM((_NBUF * _BUF_ROWS, channels), dtype),
            pltpu.SemaphoreType.DMA((_NBUF,)),
            pltpu.SemaphoreType.DMA((_NBUF,)),
        ],
    )


def kernel(x, W):
    seq_len = x.shape[1]
    k = _build(seq_len, W.shape[1], W.dtype.name)
    return k(W)


# speedup vs baseline: 1.0103x; 1.0103x over previous
import functools

import jax
import jax.numpy as jnp
from jax import lax
from jax.experimental import pallas as pl
from jax.experimental.pallas import tpu as pltpu
from jax.experimental.pallas import tpu_sc as plsc

_BUF_ROWS = 56
_NBUF = 2


@functools.lru_cache(maxsize=None)
def _build(seq_len: int, channels: int, dtype_name: str):
    info = plsc.get_sparse_core_info()
    nw = info.num_cores * info.num_subcores
    assert seq_len % nw == 0
    rows_per_w = seq_len // nw
    sizes = []
    left = rows_per_w
    while left > 0:
        c = min(_BUF_ROWS, left)
        sizes.append(c)
        left -= c
    starts = [sum(sizes[:i]) for i in range(len(sizes))]
    nchunks = len(sizes)
    dtype = jnp.dtype(dtype_name)
    mesh = plsc.VectorSubcoreMesh(core_axis_name="c", subcore_axis_name="s")

    def body(w_hbm, out_hbm, buf, load_sem, store_sem):
        wid = lax.axis_index("s") * info.num_cores + lax.axis_index("c")
        base = wid * rows_per_w

        def load(i, b):
            return pltpu.async_copy(
                w_hbm.at[pl.ds(base + starts[i], sizes[i])],
                buf.at[pl.ds(b * _BUF_ROWS, sizes[i])],
                load_sem.at[b])

        def store(i, b):
            return pltpu.async_copy(
                buf.at[pl.ds(b * _BUF_ROWS, sizes[i])],
                out_hbm.at[pl.ds(base + starts[i], sizes[i])],
                store_sem.at[b])

        loads = [None] * _NBUF
        stores = [None] * _NBUF
        loads[0] = load(0, 0)
        for i in range(nchunks):
            b = i % _NBUF
            nb = (i + 1) % _NBUF
            if i + 1 < nchunks:
                if stores[nb] is not None:
                    stores[nb].wait()
                loads[nb] = load(i + 1, nb)
            loads[b].wait()
            stores[b] = store(i, b)
        for b in range(_NBUF):
            if stores[b] is not None:
                stores[b].wait()

    return pl.kernel(
        body,
        out_type=jax.ShapeDtypeStruct((seq_len, channels), dtype),
        mesh=mesh,
        scratch_types=[
            pltpu.VMEM((_NBUF * _BUF_ROWS, channels), dtype),
            pltpu.SemaphoreType.DMA((_NBUF,)),
            pltpu.SemaphoreType.DMA((_NBUF,)),
        ],
    )


def kernel(x, W):
    seq_len = x.shape[1]
    k = _build(seq_len, W.shape[1], W.dtype.name)
    return k(W)
